# baseline (speedup 1.0000x reference)
import dataclasses
import functools

import jax
import jax.numpy as jnp
from jax import lax
from jax.experimental import pallas as pl
from jax.experimental.pallas import tpu as pltpu
from jax.experimental.pallas import tpu_sc as plsc

NC, NS, L = 2, 16, 16
NW = NC * NS
BATCH = 16384
D = 64
BPW = BATCH // NW
CHUNKS = D // L

_mesh = plsc.VectorSubcoreMesh(
    core_axis_name="c", subcore_axis_name="s", num_cores=NC, num_subcores=NS
)

_cp = pltpu.CompilerParams()
if "needs_layout_passes" in pltpu.CompilerParams.__dataclass_fields__:
    _cp = dataclasses.replace(_cp, needs_layout_passes=False)
if "use_tc_tiling_on_sc" in pltpu.CompilerParams.__dataclass_fields__:
    _cp = dataclasses.replace(_cp, use_tc_tiling_on_sc=False)


@functools.partial(
    pl.kernel,
    out_type=jax.ShapeDtypeStruct((BATCH,), jnp.float32),
    mesh=_mesh,
    scratch_types=[
        pltpu.VMEM((BPW,), jnp.int32),
        pltpu.VMEM((BPW,), jnp.int32),
        pltpu.VMEM((BPW, D), jnp.float32),
        pltpu.VMEM((BPW, D), jnp.float32),
        pltpu.VMEM((BPW,), jnp.float32),
        pltpu.VMEM((L, L + 1), jnp.float32),
        pltpu.SemaphoreType.DMA,
        pltpu.SemaphoreType.DMA,
    ],
    compiler_params=_cp,
)
def _mf_sc_kernel(users_hbm, items_hbm, eu_hbm, ei_hbm, out_hbm,
                  uidx_v, iidx_v, urows_v, irows_v, out_v, stage_v,
                  sem_u, sem_i):
    wid = lax.axis_index("s") * NC + lax.axis_index("c")
    base = wid * BPW

    pltpu.sync_copy(users_hbm.at[pl.ds(base, BPW)], uidx_v)
    pltpu.sync_copy(items_hbm.at[pl.ds(base, BPW)], iidx_v)
    cu = pltpu.async_copy(eu_hbm.at[uidx_v], urows_v, sem_u)
    ci = pltpu.async_copy(ei_hbm.at[iidx_v], irows_v, sem_i)
    cu.wait()
    ci.wait()

    row_ids = lax.iota(jnp.int32, L)
    col_ids = jnp.full((L,), L - 1, jnp.int32)

    @pl.loop(0, BPW, step=L)
    def _(r0):
        for j in range(L):
            r = r0 + j
            acc = urows_v[r, pl.ds(0, L)] * irows_v[r, pl.ds(0, L)]
            for c in range(1, CHUNKS):
                acc = acc + urows_v[r, pl.ds(c * L, L)] * irows_v[r, pl.ds(c * L, L)]
            stage_v[j, pl.ds(0, L)] = jnp.cumsum(acc)
        out_v[pl.ds(r0, L)] = plsc.load_gather(stage_v, [row_ids, col_ids])

    pltpu.sync_copy(out_v, out_hbm.at[pl.ds(base, BPW)])


def kernel(users, items, embed_user, embed_item):
    return _mf_sc_kernel(
        users.astype(jnp.int32), items.astype(jnp.int32), embed_user, embed_item
    )


# speedup vs baseline: 1.4589x; 1.4589x over previous
import dataclasses
import functools

import jax
import jax.numpy as jnp
from jax import lax
from jax.experimental import pallas as pl
from jax.experimental.pallas import tpu as pltpu
from jax.experimental.pallas import tpu_sc as plsc

NC, NS, L = 2, 16, 16
NW = NC * NS
BATCH = 16384
D = 64
BPW = BATCH // NW
CH = 32
NCHUNK = BPW // CH
G = 8

_mesh = plsc.VectorSubcoreMesh(
    core_axis_name="c", subcore_axis_name="s", num_cores=NC, num_subcores=NS
)

_cp = pltpu.CompilerParams()
if "needs_layout_passes" in pltpu.CompilerParams.__dataclass_fields__:
    _cp = dataclasses.replace(_cp, needs_layout_passes=False)


@functools.partial(
    pl.kernel,
    out_type=jax.ShapeDtypeStruct((BATCH,), jnp.float32),
    mesh=_mesh,
    scratch_types=[
        pltpu.SMEM((BPW,), jnp.int32),
        pltpu.SMEM((BPW,), jnp.int32),
        pltpu.VMEM_SHARED((NS, BPW), jnp.int32),
        pltpu.VMEM_SHARED((NS, BPW), jnp.int32),
        pltpu.VMEM((CH, G, D), jnp.float32),
        pltpu.VMEM((CH, G, D), jnp.float32),
        pltpu.VMEM((BPW,), jnp.float32),
        pltpu.VMEM((L, L + 1), jnp.float32),
        pltpu.SemaphoreType.DMA,
        pltpu.SemaphoreType.DMA,
    ],
    compiler_params=_cp,
)
def _mf_sc_kernel(users_hbm, items_hbm, eu_hbm, ei_hbm, out_hbm,
                  uidx_s, iidx_s, ush_v, ish_v, ugrp_v, igrp_v, out_v, stage_v,
                  sem_u, sem_i):
    cid = lax.axis_index("c")
    sid = lax.axis_index("s")
    wid = sid * NC + cid
    base = wid * BPW

    pltpu.sync_copy(users_hbm.at[pl.ds(base, BPW)], ush_v.at[sid])
    pltpu.sync_copy(items_hbm.at[pl.ds(base, BPW)], ish_v.at[sid])
    pltpu.sync_copy(ush_v.at[sid], uidx_s)
    pltpu.sync_copy(ish_v.at[sid], iidx_s)

    row_ids = lax.iota(jnp.int32, L)
    col_ids = jnp.full((L,), L - 1, jnp.int32)

    @pl.loop(0, NCHUNK)
    def _(t):
        t0 = t * CH

        copies = []
        for n in range(CH):
            gu = uidx_s[t0 + n] >> 3
            gi = iidx_s[t0 + n] >> 3
            copies.append(
                pltpu.async_copy(eu_hbm.at[pl.ds(gu * G, G)], ugrp_v.at[n], sem_u))
            copies.append(
                pltpu.async_copy(ei_hbm.at[pl.ds(gi * G, G)], igrp_v.at[n], sem_i))
        for cpy in copies:
            cpy.wait()

        @pl.loop(0, CH, step=L)
        def _(r0):
            for j in range(L):
                r = r0 + j
                su = uidx_s[t0 + r] & 7
                si = iidx_s[t0 + r] & 7
                acc = ugrp_v[r, su, pl.ds(0, L)] * igrp_v[r, si, pl.ds(0, L)]
                for c in range(1, D // L):
                    acc = acc + ugrp_v[r, su, pl.ds(c * L, L)] * igrp_v[r, si, pl.ds(c * L, L)]
                stage_v[j, pl.ds(0, L)] = jnp.cumsum(acc)
            out_v[pl.ds(t0 + r0, L)] = plsc.load_gather(stage_v, [row_ids, col_ids])

    pltpu.sync_copy(out_v, out_hbm.at[pl.ds(base, BPW)])


def kernel(users, items, embed_user, embed_item):
    return _mf_sc_kernel(
        users.astype(jnp.int32), items.astype(jnp.int32), embed_user, embed_item
    )
